# Initial kernel scaffold; baseline (speedup 1.0000x reference)
import jax
import jax.numpy as jnp
from jax.experimental import pallas as pl


def kernel(weights, prior_samples, x):
    raise NotImplementedError("write your pallas kernel here")


# speedup vs baseline: 7.7804x; 7.7804x over previous
import functools

import jax
import jax.numpy as jnp
from jax import lax
from jax.experimental import pallas as pl
from jax.experimental.pallas import tpu as pltpu
from jax.experimental.pallas import tpu_sc as plsc

N_BINS = 1024
N = 4194304
NQ = 131072
NW = 32
EPW = N // NW
QPW = NQ // NW
L = 16
CH = 8192
NCH = EPW // CH

H1 = 65536
R2, R3, R4 = 32, 32, 64
H2 = N_BINS * R2
H3 = N_BINS * R3
H4 = N_BINS * R4

_I32_MIN = jnp.int32(-2147483648)
_M31 = jnp.int32(0x7FFFFFFF)


def _wid():
    return lax.axis_index("s") * 2 + lax.axis_index("c")


def _ukey(vf):
    u = plsc.bitcast(vf, jnp.int32)
    s = lax.shift_right_arithmetic(u, 31)
    key = u ^ (s & _M31)
    return key ^ _I32_MIN


def _zero_hist(hist, n):
    zero = jnp.zeros((L,), jnp.int32)

    def body(i, carry):
        hist[pl.ds(i * L, L)] = zero
        return carry

    lax.fori_loop(0, n // L, body, 0)


@functools.cache
def _build():
    mesh = plsc.VectorSubcoreMesh(
        core_axis_name="c", subcore_axis_name="s",
        num_cores=2, num_subcores=16)

    @functools.partial(
        pl.kernel,
        out_type=jax.ShapeDtypeStruct((NW * H1,), jnp.int32),
        mesh=mesh,
        compiler_params=pltpu.CompilerParams(needs_layout_passes=False),
        scratch_types=[
            pltpu.VMEM((H1,), jnp.int32),
            pltpu.VMEM((CH,), jnp.float32),
        ],
    )
    def pass1(d_hbm, out_hbm, hist, buf):
        wid = _wid()
        _zero_hist(hist, H1)
        base = wid * EPW

        def chunk(c, carry):
            pltpu.sync_copy(d_hbm.at[pl.ds(base + c * CH, CH)], buf)

            def vec(i, carry2):
                uk = _ukey(buf[pl.ds(i * L, L)])
                b = lax.shift_right_logical(uk, 16)
                cnt, last = plsc.scan_count(b)
                plsc.addupdate_scatter(hist, [b], cnt, mask=last)
                return carry2

            lax.fori_loop(0, CH // L, vec, 0)
            return carry

        lax.fori_loop(0, NCH, chunk, 0)
        pltpu.sync_copy(hist, out_hbm.at[pl.ds(wid * H1, H1)])

    @functools.partial(
        pl.kernel,
        out_type=(
            jax.ShapeDtypeStruct((NW * H2,), jnp.int32),
            jax.ShapeDtypeStruct((N,), jnp.int32),
        ),
        mesh=mesh,
        compiler_params=pltpu.CompilerParams(needs_layout_passes=False),
        scratch_types=[
            pltpu.VMEM((H1,), jnp.int32),
            pltpu.VMEM((H2,), jnp.int32),
            pltpu.VMEM((CH,), jnp.float32),
            pltpu.VMEM((CH,), jnp.int32),
        ],
    )
    def pass2(d_hbm, lut1_hbm, hist_hbm, aux_hbm, lut1, hist, buf, abuf):
        wid = _wid()
        pltpu.sync_copy(lut1_hbm, lut1)
        _zero_hist(hist, H2)
        base = wid * EPW

        def chunk(c, carry):
            pltpu.sync_copy(d_hbm.at[pl.ds(base + c * CH, CH)], buf)

            def vec(i, carry2):
                uk = _ukey(buf[pl.ds(i * L, L)])
                b = lax.shift_right_logical(uk, 16)
                row = plsc.load_gather(lut1, [b])
                valid = row >= 0
                bits = lax.shift_right_logical(uk, 11) & jnp.int32(R2 - 1)
                bn = jnp.where(valid, row * R2 + bits, jnp.int32(0))
                cnt, last = plsc.scan_count(bn, mask=valid)
                plsc.addupdate_scatter(hist, [bn], cnt, mask=last)
                abuf[pl.ds(i * L, L)] = jnp.where(valid, bn, jnp.int32(-1))
                return carry2

            lax.fori_loop(0, CH // L, vec, 0)
            pltpu.sync_copy(abuf, aux_hbm.at[pl.ds(base + c * CH, CH)])
            return carry

        lax.fori_loop(0, NCH, chunk, 0)
        pltpu.sync_copy(hist, hist_hbm.at[pl.ds(wid * H2, H2)])

    def refine_pass(shift, radix, lut_size, hist_size, write_aux):
        out_type = jax.ShapeDtypeStruct((NW * hist_size,), jnp.int32)
        scratch = [
            pltpu.VMEM((lut_size,), jnp.int32),
            pltpu.VMEM((hist_size,), jnp.int32),
            pltpu.VMEM((CH,), jnp.float32),
            pltpu.VMEM((CH,), jnp.int32),
        ]
        if write_aux:
            out_type = (out_type, jax.ShapeDtypeStruct((N,), jnp.int32))
            scratch = scratch + [pltpu.VMEM((CH,), jnp.int32)]

        def body(d_hbm, aux_in_hbm, lut_hbm, *refs):
            if write_aux:
                hist_hbm, aux_hbm, lut, hist, buf, abuf, obuf = refs
            else:
                hist_hbm, lut, hist, buf, abuf = refs
            wid = _wid()
            pltpu.sync_copy(lut_hbm, lut)
            _zero_hist(hist, hist_size)
            base = wid * EPW

            def chunk(c, carry):
                pltpu.sync_copy(d_hbm.at[pl.ds(base + c * CH, CH)], buf)
                pltpu.sync_copy(aux_in_hbm.at[pl.ds(base + c * CH, CH)], abuf)

                def vec(i, carry2):
                    uk = _ukey(buf[pl.ds(i * L, L)])
                    av = abuf[pl.ds(i * L, L)]
                    idx = jnp.where(av < 0, jnp.int32(0), av)
                    row = plsc.load_gather(lut, [idx])
                    valid = (av >= 0) & (row >= 0)
                    bits = (lax.shift_right_logical(uk, shift)
                            & jnp.int32(radix - 1))
                    bn = jnp.where(valid, row * radix + bits, jnp.int32(0))
                    cnt, last = plsc.scan_count(bn, mask=valid)
                    plsc.addupdate_scatter(hist, [bn], cnt, mask=last)
                    if write_aux:
                        obuf[pl.ds(i * L, L)] = jnp.where(
                            valid, bn, jnp.int32(-1))
                    return carry2

                lax.fori_loop(0, CH // L, vec, 0)
                if write_aux:
                    pltpu.sync_copy(obuf, aux_hbm.at[pl.ds(base + c * CH, CH)])
                return carry

            lax.fori_loop(0, NCH, chunk, 0)
            pltpu.sync_copy(hist,
                            hist_hbm.at[pl.ds(wid * hist_size, hist_size)])

        return functools.partial(
            pl.kernel, out_type=out_type, mesh=mesh,
            compiler_params=pltpu.CompilerParams(needs_layout_passes=False),
            scratch_types=scratch)(body)

    pass3 = refine_pass(shift=6, radix=R3, lut_size=H2, hist_size=H3,
                        write_aux=True)
    pass4 = refine_pass(shift=0, radix=R4, lut_size=H3, hist_size=H4,
                        write_aux=False)

    @functools.partial(
        pl.kernel,
        out_type=jax.ShapeDtypeStruct((NW * 6 * L,), jnp.float32),
        mesh=mesh,
        compiler_params=pltpu.CompilerParams(needs_layout_passes=False),
        scratch_types=[
            pltpu.VMEM((CH,), jnp.float32),
            pltpu.VMEM((6 * L,), jnp.float32),
            pltpu.VMEM((2 * L,), jnp.float32),
        ],
    )
    def pass5(d_hbm, params_hbm, out_hbm, buf, obuf, pbuf):
        wid = _wid()
        pltpu.sync_copy(params_hbm, pbuf)
        fb0 = pbuf[pl.ds(0, L)]
        fbl = pbuf[pl.ds(L, L)]
        base = wid * EPW
        zf = jnp.zeros((L,), jnp.float32)

        def chunk(c, carry):
            pltpu.sync_copy(d_hbm.at[pl.ds(base + c * CH, CH)], buf)

            def vec(i, acc):
                cl, sl, ql, cr, sr, qr = acc
                v = buf[pl.ds(i * L, L)]
                ml = v < fb0
                mr = v >= fbl
                dl = v - fb0
                dr = v - fbl
                zero = jnp.float32(0.0)
                cl = cl + jnp.where(ml, jnp.float32(1.0), zero)
                sl = sl + jnp.where(ml, dl, zero)
                ql = ql + jnp.where(ml, dl * dl, zero)
                cr = cr + jnp.where(mr, jnp.float32(1.0), zero)
                sr = sr + jnp.where(mr, dr, zero)
                qr = qr + jnp.where(mr, dr * dr, zero)
                return (cl, sl, ql, cr, sr, qr)

            return lax.fori_loop(0, CH // L, vec, carry)

        acc = lax.fori_loop(0, NCH, chunk, (zf, zf, zf, zf, zf, zf))
        for k in range(6):
            obuf[pl.ds(k * L, L)] = acc[k]
        pltpu.sync_copy(obuf, out_hbm.at[pl.ds(wid * 6 * L, 6 * L)])

    @functools.partial(
        pl.kernel,
        out_type=jax.ShapeDtypeStruct((NQ,), jnp.float32),
        mesh=mesh,
        compiler_params=pltpu.CompilerParams(needs_layout_passes=False),
        scratch_types=[
            pltpu.VMEM((N_BINS,), jnp.float32),
            pltpu.VMEM((N_BINS,), jnp.float32),
            pltpu.VMEM((6 * L,), jnp.float32),
            pltpu.VMEM((QPW,), jnp.float32),
            pltpu.VMEM((QPW,), jnp.float32),
        ],
    )
    def pass6(x_hbm, fb_hbm, w_hbm, params_hbm, out_hbm,
              fb, w, prm, qbuf, obuf):
        wid = _wid()
        pltpu.sync_copy(fb_hbm, fb)
        pltpu.sync_copy(w_hbm, w)
        pltpu.sync_copy(params_hbm, prm)
        base = wid * QPW
        pltpu.sync_copy(x_hbm.at[pl.ds(base, QPW)], qbuf)
        fb0 = prm[pl.ds(0, L)]
        fbl = prm[pl.ds(L, L)]
        c_l = prm[pl.ds(2 * L, L)]
        ils = prm[pl.ds(3 * L, L)]
        c_r = prm[pl.ds(4 * L, L)]
        irs = prm[pl.ds(5 * L, L)]
        inf = jnp.float32(jnp.inf)

        def vec(i, carry):
            xq = qbuf[pl.ds(i * L, L)]
            pos = jnp.zeros((L,), jnp.int32)
            for step in (512, 256, 128, 64, 32, 16, 8, 4, 2, 1):
                cand = pos + jnp.int32(step)
                vb = plsc.load_gather(fb, [cand - 1])
                pos = jnp.where(vb <= xq, cand, pos)
            wv = plsc.load_gather(w, [pos])
            hi = plsc.load_gather(fb, [pos])
            lo_idx = jnp.maximum(pos - 1, jnp.int32(0))
            lov = plsc.load_gather(fb, [lo_idx])
            lov = jnp.where(pos == 0, -inf, lov)
            density = wv / (hi - lov)
            tl = (fb0 - xq) * ils
            tr = (xq - fbl) * irs
            left = c_l * jnp.exp(jnp.float32(-0.5) * tl * tl)
            right = c_r * jnp.exp(jnp.float32(-0.5) * tr * tr)
            half = jnp.where(xq < fb0, left, right)
            mid = (xq >= fb0) & (xq < fbl)
            obuf[pl.ds(i * L, L)] = jnp.where(mid, density, half)
            return carry

        lax.fori_loop(0, QPW // L, vec, 0)
        pltpu.sync_copy(obuf, out_hbm.at[pl.ds(base, QPW)])

    return pass1, pass2, pass3, pass4, pass5, pass6


def _rows_and_lut(bins, table_size):
    isnew = jnp.concatenate(
        [jnp.ones((1,), jnp.bool_), bins[1:] != bins[:-1]])
    rows = jnp.cumsum(isnew.astype(jnp.int32)) - 1
    lut = jnp.full((table_size,), -1, jnp.int32).at[bins].set(rows)
    return rows, lut


def _split_ranks(hist2d, rows, rw):
    h = hist2d[rows]
    c = jnp.cumsum(h, axis=1)
    digit = jnp.sum((c <= rw[:, None]).astype(jnp.int32), axis=1)
    excl = c - h
    rw_new = rw - jnp.take_along_axis(excl, digit[:, None], axis=1)[:, 0]
    return digit.astype(jnp.int32), rw_new


def kernel(weights, prior_samples, x):
    pass1, pass2, pass3, pass4, pass5, pass6 = _build()
    d = prior_samples
    ranks = jnp.int32(N // N_BINS) * jnp.arange(1, N_BINS, dtype=jnp.int32)

    hist1 = pass1(d).reshape(NW, H1).sum(axis=0)
    cdf1 = jnp.cumsum(hist1)
    b16 = jnp.searchsorted(cdf1, ranks, side="right").astype(jnp.int32)
    rw = ranks - (cdf1[b16] - hist1[b16])
    rows1, lut1 = _rows_and_lut(b16, H1)

    hist2, aux2 = pass2(d, lut1)
    bits2, rw = _split_ranks(hist2.reshape(NW, N_BINS, R2).sum(axis=0),
                             rows1, rw)
    bin2 = rows1 * R2 + bits2
    rows2, lut2 = _rows_and_lut(bin2, H2)

    hist3, aux3 = pass3(d, aux2, lut2)
    bits3, rw = _split_ranks(hist3.reshape(NW, N_BINS, R3).sum(axis=0),
                             rows2, rw)
    bin3 = rows2 * R3 + bits3
    rows3, lut3 = _rows_and_lut(bin3, H3)

    hist4 = pass4(d, aux3, lut3)
    bits4, _ = _split_ranks(hist4.reshape(NW, N_BINS, R4).sum(axis=0),
                            rows3, rw)

    uk = (b16 << 16) | (bits2 << 11) | (bits3 << 6) | bits4
    key = uk ^ _I32_MIN
    ubits = key ^ (jnp.right_shift(key, 31) & _M31)
    fb = lax.bitcast_convert_type(ubits, jnp.float32)

    fb0 = fb[0]
    fbl = fb[-1]
    p5 = jnp.broadcast_to(jnp.stack([fb0, fbl])[:, None], (2, L)).reshape(-1)
    parts = pass5(d, p5).reshape(NW, 6, L).sum(axis=(0, 2))
    cl, sl, ql, cr, sr, qr = (parts[i] for i in range(6))
    scale = 1.0 / jnp.sqrt(jnp.float32(1.0 - 2.0 / jnp.pi))
    ls = jnp.sqrt(ql / cl - (sl / cl) ** 2) * scale
    rs = jnp.sqrt(qr / cr - (sr / cr) ** 2) * scale

    w = weights / weights.sum()
    s2pi = jnp.sqrt(jnp.float32(2.0) * jnp.pi)
    c_l = w[0] * 2.0 / (ls * s2pi)
    c_r = w[-1] * 2.0 / (rs * s2pi)
    prm = jnp.broadcast_to(
        jnp.stack([fb0, fbl, c_l, 1.0 / ls, c_r, 1.0 / rs])[:, None],
        (6, L)).reshape(-1)
    fbpad = jnp.concatenate([fb, jnp.full((1,), jnp.inf, jnp.float32)])
    return pass6(x, fbpad, w, prm)


# speedup vs baseline: 8.0055x; 1.0289x over previous
import functools

import jax
import jax.numpy as jnp
from jax import lax
from jax.experimental import pallas as pl
from jax.experimental.pallas import tpu as pltpu
from jax.experimental.pallas import tpu_sc as plsc

N_BINS = 1024
N = 4194304
NQ = 131072
NW = 32
EPW = N // NW
QPW = NQ // NW
L = 16
CH = 8192
NCH = EPW // CH
UNR = 8

H1 = 65536
R2, R3, R4 = 32, 32, 64
H2 = N_BINS * R2
H3 = N_BINS * R3
H4 = N_BINS * R4

_I32_MIN = jnp.int32(-2147483648)
_M31 = jnp.int32(0x7FFFFFFF)


def _wid():
    return lax.axis_index("s") * 2 + lax.axis_index("c")


def _ukey(vf):
    u = plsc.bitcast(vf, jnp.int32)
    s = lax.shift_right_arithmetic(u, 31)
    key = u ^ (s & _M31)
    return key ^ _I32_MIN


def _zero_hist(hist, n):
    zero = jnp.zeros((L,), jnp.int32)

    def body(i, carry):
        hist[pl.ds(i * L, L)] = zero
        return carry

    lax.fori_loop(0, n // L, body, 0)


@functools.cache
def _build():
    mesh = plsc.VectorSubcoreMesh(
        core_axis_name="c", subcore_axis_name="s",
        num_cores=2, num_subcores=16)

    @functools.partial(
        pl.kernel,
        out_type=jax.ShapeDtypeStruct((NW * H1,), jnp.int32),
        mesh=mesh,
        compiler_params=pltpu.CompilerParams(needs_layout_passes=False),
        scratch_types=[
            pltpu.VMEM((H1,), jnp.int32),
            pltpu.VMEM((CH,), jnp.float32),
        ],
    )
    def pass1(d_hbm, out_hbm, hist, buf):
        wid = _wid()
        _zero_hist(hist, H1)
        base = wid * EPW

        def chunk(c, carry):
            pltpu.sync_copy(d_hbm.at[pl.ds(base + c * CH, CH)], buf)

            def vec(i, carry2):
                for j in range(UNR):
                    uk = _ukey(buf[pl.ds((i * UNR + j) * L, L)])
                    b = lax.shift_right_logical(uk, 16)
                    cnt, last = plsc.scan_count(b)
                    plsc.addupdate_scatter(hist, [b], cnt, mask=last)
                return carry2

            lax.fori_loop(0, CH // (L * UNR), vec, 0)
            return carry

        lax.fori_loop(0, NCH, chunk, 0)
        pltpu.sync_copy(hist, out_hbm.at[pl.ds(wid * H1, H1)])

    @functools.partial(
        pl.kernel,
        out_type=(
            jax.ShapeDtypeStruct((NW * H2,), jnp.int32),
            jax.ShapeDtypeStruct((N,), jnp.int32),
        ),
        mesh=mesh,
        compiler_params=pltpu.CompilerParams(needs_layout_passes=False),
        scratch_types=[
            pltpu.VMEM((H1,), jnp.int32),
            pltpu.VMEM((H2,), jnp.int32),
            pltpu.VMEM((CH,), jnp.float32),
            pltpu.VMEM((CH,), jnp.int32),
        ],
    )
    def pass2(d_hbm, lut1_hbm, hist_hbm, aux_hbm, lut1, hist, buf, abuf):
        wid = _wid()
        pltpu.sync_copy(lut1_hbm, lut1)
        _zero_hist(hist, H2)
        base = wid * EPW

        def chunk(c, carry):
            pltpu.sync_copy(d_hbm.at[pl.ds(base + c * CH, CH)], buf)

            def vec(i, carry2):
                for j in range(UNR):
                    o = (i * UNR + j) * L
                    uk = _ukey(buf[pl.ds(o, L)])
                    b = lax.shift_right_logical(uk, 16)
                    row = plsc.load_gather(lut1, [b])
                    valid = row >= 0
                    bits = lax.shift_right_logical(uk, 11) & jnp.int32(R2 - 1)
                    bn = jnp.where(valid, row * R2 + bits, jnp.int32(0))
                    cnt, last = plsc.scan_count(bn, mask=valid)
                    plsc.addupdate_scatter(hist, [bn], cnt, mask=last)
                    abuf[pl.ds(o, L)] = jnp.where(valid, bn, jnp.int32(-1))
                return carry2

            lax.fori_loop(0, CH // (L * UNR), vec, 0)
            pltpu.sync_copy(abuf, aux_hbm.at[pl.ds(base + c * CH, CH)])
            return carry

        lax.fori_loop(0, NCH, chunk, 0)
        pltpu.sync_copy(hist, hist_hbm.at[pl.ds(wid * H2, H2)])

    def refine_pass(shift, radix, lut_size, hist_size, write_aux):
        out_type = jax.ShapeDtypeStruct((NW * hist_size,), jnp.int32)
        scratch = [
            pltpu.VMEM((lut_size,), jnp.int32),
            pltpu.VMEM((hist_size,), jnp.int32),
            pltpu.VMEM((CH,), jnp.float32),
            pltpu.VMEM((CH,), jnp.int32),
        ]
        if write_aux:
            out_type = (out_type, jax.ShapeDtypeStruct((N,), jnp.int32))
            scratch = scratch + [pltpu.VMEM((CH,), jnp.int32)]

        def body(d_hbm, aux_in_hbm, lut_hbm, *refs):
            if write_aux:
                hist_hbm, aux_hbm, lut, hist, buf, abuf, obuf = refs
            else:
                hist_hbm, lut, hist, buf, abuf = refs
            wid = _wid()
            pltpu.sync_copy(lut_hbm, lut)
            _zero_hist(hist, hist_size)
            base = wid * EPW

            def chunk(c, carry):
                pltpu.sync_copy(d_hbm.at[pl.ds(base + c * CH, CH)], buf)
                pltpu.sync_copy(aux_in_hbm.at[pl.ds(base + c * CH, CH)], abuf)

                def vec(i, carry2):
                    for j in range(UNR):
                        o = (i * UNR + j) * L
                        uk = _ukey(buf[pl.ds(o, L)])
                        av = abuf[pl.ds(o, L)]
                        idx = jnp.where(av < 0, jnp.int32(0), av)
                        row = plsc.load_gather(lut, [idx])
                        valid = (av >= 0) & (row >= 0)
                        bits = (lax.shift_right_logical(uk, shift)
                                & jnp.int32(radix - 1))
                        bn = jnp.where(valid, row * radix + bits, jnp.int32(0))
                        cnt, last = plsc.scan_count(bn, mask=valid)
                        plsc.addupdate_scatter(hist, [bn], cnt, mask=last)
                        if write_aux:
                            obuf[pl.ds(o, L)] = jnp.where(
                                valid, bn, jnp.int32(-1))
                    return carry2

                lax.fori_loop(0, CH // (L * UNR), vec, 0)
                if write_aux:
                    pltpu.sync_copy(obuf, aux_hbm.at[pl.ds(base + c * CH, CH)])
                return carry

            lax.fori_loop(0, NCH, chunk, 0)
            pltpu.sync_copy(hist,
                            hist_hbm.at[pl.ds(wid * hist_size, hist_size)])

        return functools.partial(
            pl.kernel, out_type=out_type, mesh=mesh,
            compiler_params=pltpu.CompilerParams(needs_layout_passes=False),
            scratch_types=scratch)(body)

    pass3 = refine_pass(shift=6, radix=R3, lut_size=H2, hist_size=H3,
                        write_aux=True)
    pass4 = refine_pass(shift=0, radix=R4, lut_size=H3, hist_size=H4,
                        write_aux=False)

    @functools.partial(
        pl.kernel,
        out_type=jax.ShapeDtypeStruct((NW * 6 * L,), jnp.float32),
        mesh=mesh,
        compiler_params=pltpu.CompilerParams(needs_layout_passes=False),
        scratch_types=[
            pltpu.VMEM((CH,), jnp.float32),
            pltpu.VMEM((6 * L,), jnp.float32),
            pltpu.VMEM((2 * L,), jnp.float32),
        ],
    )
    def pass5(d_hbm, params_hbm, out_hbm, buf, obuf, pbuf):
        wid = _wid()
        pltpu.sync_copy(params_hbm, pbuf)
        fb0 = pbuf[pl.ds(0, L)]
        fbl = pbuf[pl.ds(L, L)]
        base = wid * EPW
        zf = jnp.zeros((L,), jnp.float32)

        def chunk(c, carry):
            pltpu.sync_copy(d_hbm.at[pl.ds(base + c * CH, CH)], buf)

            def vec(i, acc):
                cl, sl, ql, cr, sr, qr = acc
                zero = jnp.float32(0.0)
                for j in range(UNR):
                    v = buf[pl.ds((i * UNR + j) * L, L)]
                    ml = v < fb0
                    mr = v >= fbl
                    dl = v - fb0
                    dr = v - fbl
                    cl = cl + jnp.where(ml, jnp.float32(1.0), zero)
                    sl = sl + jnp.where(ml, dl, zero)
                    ql = ql + jnp.where(ml, dl * dl, zero)
                    cr = cr + jnp.where(mr, jnp.float32(1.0), zero)
                    sr = sr + jnp.where(mr, dr, zero)
                    qr = qr + jnp.where(mr, dr * dr, zero)
                return (cl, sl, ql, cr, sr, qr)

            return lax.fori_loop(0, CH // (L * UNR), vec, carry)

        acc = lax.fori_loop(0, NCH, chunk, (zf, zf, zf, zf, zf, zf))
        for k in range(6):
            obuf[pl.ds(k * L, L)] = acc[k]
        pltpu.sync_copy(obuf, out_hbm.at[pl.ds(wid * 6 * L, 6 * L)])

    @functools.partial(
        pl.kernel,
        out_type=jax.ShapeDtypeStruct((NQ,), jnp.float32),
        mesh=mesh,
        compiler_params=pltpu.CompilerParams(needs_layout_passes=False),
        scratch_types=[
            pltpu.VMEM((N_BINS,), jnp.float32),
            pltpu.VMEM((N_BINS,), jnp.float32),
            pltpu.VMEM((6 * L,), jnp.float32),
            pltpu.VMEM((QPW,), jnp.float32),
            pltpu.VMEM((QPW,), jnp.float32),
        ],
    )
    def pass6(x_hbm, fb_hbm, w_hbm, params_hbm, out_hbm,
              fb, w, prm, qbuf, obuf):
        wid = _wid()
        pltpu.sync_copy(fb_hbm, fb)
        pltpu.sync_copy(w_hbm, w)
        pltpu.sync_copy(params_hbm, prm)
        base = wid * QPW
        pltpu.sync_copy(x_hbm.at[pl.ds(base, QPW)], qbuf)
        fb0 = prm[pl.ds(0, L)]
        fbl = prm[pl.ds(L, L)]
        c_l = prm[pl.ds(2 * L, L)]
        ils = prm[pl.ds(3 * L, L)]
        c_r = prm[pl.ds(4 * L, L)]
        irs = prm[pl.ds(5 * L, L)]
        inf = jnp.float32(jnp.inf)

        def vec(i, carry):
            xq = qbuf[pl.ds(i * L, L)]
            pos = jnp.zeros((L,), jnp.int32)
            for step in (512, 256, 128, 64, 32, 16, 8, 4, 2, 1):
                cand = pos + jnp.int32(step)
                vb = plsc.load_gather(fb, [cand - 1])
                pos = jnp.where(vb <= xq, cand, pos)
            wv = plsc.load_gather(w, [pos])
            hi = plsc.load_gather(fb, [pos])
            lo_idx = jnp.maximum(pos - 1, jnp.int32(0))
            lov = plsc.load_gather(fb, [lo_idx])
            lov = jnp.where(pos == 0, -inf, lov)
            density = wv / (hi - lov)
            tl = (fb0 - xq) * ils
            tr = (xq - fbl) * irs
            left = c_l * jnp.exp(jnp.float32(-0.5) * tl * tl)
            right = c_r * jnp.exp(jnp.float32(-0.5) * tr * tr)
            half = jnp.where(xq < fb0, left, right)
            mid = (xq >= fb0) & (xq < fbl)
            obuf[pl.ds(i * L, L)] = jnp.where(mid, density, half)
            return carry

        lax.fori_loop(0, QPW // L, vec, 0)
        pltpu.sync_copy(obuf, out_hbm.at[pl.ds(base, QPW)])

    return pass1, pass2, pass3, pass4, pass5, pass6


def _rows_and_lut(bins, table_size):
    isnew = jnp.concatenate(
        [jnp.ones((1,), jnp.bool_), bins[1:] != bins[:-1]])
    rows = jnp.cumsum(isnew.astype(jnp.int32)) - 1
    lut = jnp.full((table_size,), -1, jnp.int32).at[bins].set(rows)
    return rows, lut


def _split_ranks(hist2d, rows, rw):
    h = hist2d[rows]
    c = jnp.cumsum(h, axis=1)
    digit = jnp.sum((c <= rw[:, None]).astype(jnp.int32), axis=1)
    excl = c - h
    rw_new = rw - jnp.take_along_axis(excl, digit[:, None], axis=1)[:, 0]
    return digit.astype(jnp.int32), rw_new


def kernel(weights, prior_samples, x):
    pass1, pass2, pass3, pass4, pass5, pass6 = _build()
    d = prior_samples
    ranks = jnp.int32(N // N_BINS) * jnp.arange(1, N_BINS, dtype=jnp.int32)

    hist1 = pass1(d).reshape(NW, H1).sum(axis=0)
    cdf1 = jnp.cumsum(hist1)
    b16 = jnp.searchsorted(cdf1, ranks, side="right").astype(jnp.int32)
    rw = ranks - (cdf1[b16] - hist1[b16])
    rows1, lut1 = _rows_and_lut(b16, H1)

    hist2, aux2 = pass2(d, lut1)
    bits2, rw = _split_ranks(hist2.reshape(NW, N_BINS, R2).sum(axis=0),
                             rows1, rw)
    bin2 = rows1 * R2 + bits2
    rows2, lut2 = _rows_and_lut(bin2, H2)

    hist3, aux3 = pass3(d, aux2, lut2)
    bits3, rw = _split_ranks(hist3.reshape(NW, N_BINS, R3).sum(axis=0),
                             rows2, rw)
    bin3 = rows2 * R3 + bits3
    rows3, lut3 = _rows_and_lut(bin3, H3)

    hist4 = pass4(d, aux3, lut3)
    bits4, _ = _split_ranks(hist4.reshape(NW, N_BINS, R4).sum(axis=0),
                            rows3, rw)

    uk = (b16 << 16) | (bits2 << 11) | (bits3 << 6) | bits4
    key = uk ^ _I32_MIN
    ubits = key ^ (jnp.right_shift(key, 31) & _M31)
    fb = lax.bitcast_convert_type(ubits, jnp.float32)

    fb0 = fb[0]
    fbl = fb[-1]
    p5 = jnp.broadcast_to(jnp.stack([fb0, fbl])[:, None], (2, L)).reshape(-1)
    parts = pass5(d, p5).reshape(NW, 6, L).sum(axis=(0, 2))
    cl, sl, ql, cr, sr, qr = (parts[i] for i in range(6))
    scale = 1.0 / jnp.sqrt(jnp.float32(1.0 - 2.0 / jnp.pi))
    ls = jnp.sqrt(ql / cl - (sl / cl) ** 2) * scale
    rs = jnp.sqrt(qr / cr - (sr / cr) ** 2) * scale

    w = weights / weights.sum()
    s2pi = jnp.sqrt(jnp.float32(2.0) * jnp.pi)
    c_l = w[0] * 2.0 / (ls * s2pi)
    c_r = w[-1] * 2.0 / (rs * s2pi)
    prm = jnp.broadcast_to(
        jnp.stack([fb0, fbl, c_l, 1.0 / ls, c_r, 1.0 / rs])[:, None],
        (6, L)).reshape(-1)
    fbpad = jnp.concatenate([fb, jnp.full((1,), jnp.inf, jnp.float32)])
    return pass6(x, fbpad, w, prm)


# speedup vs baseline: 8.9227x; 1.1146x over previous
import functools

import jax
import jax.numpy as jnp
from jax import lax
from jax.experimental import pallas as pl
from jax.experimental.pallas import tpu as pltpu
from jax.experimental.pallas import tpu_sc as plsc

N_BINS = 1024
N = 4194304
NQ = 131072
NW = 32
EPW = N // NW
QPW = NQ // NW
L = 16
CH = 8192
NCH = EPW // CH
UNR = 8

B1, B2x, B3, B4 = 15, 5, 6, 6
H1 = 1 << B1
H2 = N_BINS << B2x
H3 = N_BINS << B3
H4 = N_BINS << B4
S2, S3 = 12, 6

_I32_MIN = jnp.int32(-2147483648)
_M31 = jnp.int32(0x7FFFFFFF)


def _wid():
    return lax.axis_index("s") * 2 + lax.axis_index("c")


def _ukey(vf):
    u = plsc.bitcast(vf, jnp.int32)
    s = lax.shift_right_arithmetic(u, 31)
    key = u ^ (s & _M31)
    return key ^ _I32_MIN


def _zero(ref, n):
    zv = jnp.zeros((L,), jnp.int32)

    def body(i, carry):
        ref[pl.ds(i * L, L)] = zv
        return carry

    lax.fori_loop(0, n // L, body, 0)


def _merge_store(ha, hb, n, out_hbm, offset):
    def body(i, carry):
        ha[pl.ds(i * L, L)] = ha[pl.ds(i * L, L)] + hb[pl.ds(i * L, L)]
        return carry

    lax.fori_loop(0, n // L, body, 0)
    pltpu.sync_copy(ha, out_hbm.at[pl.ds(offset, n)])


@functools.cache
def _build():
    mesh = plsc.VectorSubcoreMesh(
        core_axis_name="c", subcore_axis_name="s",
        num_cores=2, num_subcores=16)
    cparams = pltpu.CompilerParams(needs_layout_passes=False)

    @functools.partial(
        pl.kernel,
        out_type=jax.ShapeDtypeStruct((NW * H1,), jnp.int32),
        mesh=mesh,
        compiler_params=cparams,
        scratch_types=[
            pltpu.VMEM((H1,), jnp.int32),
            pltpu.VMEM((H1,), jnp.int32),
            pltpu.VMEM((CH,), jnp.float32),
        ],
    )
    def pass1(d_hbm, out_hbm, ha, hb, buf):
        wid = _wid()
        _zero(ha, H1)
        _zero(hb, H1)
        base = wid * EPW
        hists = (ha, hb)

        def chunk(c, carry):
            pltpu.sync_copy(d_hbm.at[pl.ds(base + c * CH, CH)], buf)

            def vec(i, carry2):
                for j in range(UNR):
                    uk = _ukey(buf[pl.ds((i * UNR + j) * L, L)])
                    b = lax.shift_right_logical(uk, 32 - B1)
                    cnt, last = plsc.scan_count(b)
                    plsc.addupdate_scatter(hists[j % 2], [b], cnt, mask=last)
                return carry2

            lax.fori_loop(0, CH // (L * UNR), vec, 0)
            return carry

        lax.fori_loop(0, NCH, chunk, 0)
        _merge_store(ha, hb, H1, out_hbm, wid * H1)

    @functools.partial(
        pl.kernel,
        out_type=(
            jax.ShapeDtypeStruct((NW * H2,), jnp.int32),
            jax.ShapeDtypeStruct((N,), jnp.int32),
            jax.ShapeDtypeStruct((N,), jnp.int32),
            jax.ShapeDtypeStruct((NW * L,), jnp.int32),
            jax.ShapeDtypeStruct((NW * L,), jnp.int32),
        ),
        mesh=mesh,
        compiler_params=cparams,
        scratch_types=[
            pltpu.VMEM((H1,), jnp.int32),
            pltpu.VMEM((H2,), jnp.int32),
            pltpu.VMEM((H2,), jnp.int32),
            pltpu.VMEM((CH,), jnp.float32),
            pltpu.VMEM((CH // 2 + L,), jnp.int32),
            pltpu.VMEM((CH // 2 + L,), jnp.int32),
            pltpu.VMEM((L,), jnp.int32),
        ],
    )
    def pass2(d_hbm, lut_hbm, hist_hbm, sa_hbm, sb_hbm, ca_hbm, cb_hbm,
              lut, ha, hb, buf, cba, cbb, cntv):
        wid = _wid()
        pltpu.sync_copy(lut_hbm, lut)
        _zero(ha, H2)
        _zero(hb, H2)
        base = wid * EPW
        hists = (ha, hb)
        cbufs = (cba, cbb)
        neg1 = jnp.full((L,), -1, jnp.int32)
        iota = lax.iota(jnp.int32, L)
        low_mask = jnp.int32((1 << S2) - 1)
        HC = CH // 2

        def chunk(c, totals):
            pltpu.sync_copy(d_hbm.at[pl.ds(base + c * CH, CH)], buf)

            def vec(i, offs):
                offs = list(offs)
                for j in range(UNR):
                    uk = _ukey(buf[pl.ds((i * UNR + j) * L, L)])
                    b = lax.shift_right_logical(uk, 32 - B1)
                    row = plsc.load_gather(lut, [b])
                    valid = row >= 0
                    bits = (lax.shift_right_logical(uk, S2)
                            & jnp.int32((1 << B2x) - 1))
                    bn = jnp.where(valid, (row << B2x) + bits, jnp.int32(0))
                    cnt, last = plsc.scan_count(bn, mask=valid)
                    plsc.addupdate_scatter(hists[j % 2], [bn], cnt, mask=last)
                    packed = (bn << S2) | (uk & low_mask)
                    k = j % 2
                    vi = valid.astype(jnp.int32)
                    pos = plsc.cumsum(vi)
                    dest = offs[k] + jnp.maximum(pos - 1, jnp.int32(0))
                    plsc.store_scatter(cbufs[k], [dest], packed, mask=valid)
                    offs[k] = offs[k] + jnp.sum(vi)
                return tuple(offs)

            offa, offb = lax.fori_loop(0, CH // (L * UNR), vec,
                                       (jnp.int32(0), jnp.int32(0)))
            plsc.store_scatter(cba, [offa + iota], neg1)
            plsc.store_scatter(cbb, [offb + iota], neg1)
            offa = (offa + 7) & jnp.int32(~7)
            offb = (offb + 7) & jnp.int32(~7)
            ta, tb = totals
            pltpu.sync_copy(
                cba.at[pl.ds(0, HC)],
                sa_hbm.at[pl.ds(pl.multiple_of(base + ta, 8), HC)])
            pltpu.sync_copy(
                cbb.at[pl.ds(0, HC)],
                sb_hbm.at[pl.ds(pl.multiple_of(base + tb, 8), HC)])
            return (ta + offa, tb + offb)

        ta, tb = lax.fori_loop(0, NCH, chunk, (jnp.int32(0), jnp.int32(0)))
        cntv[pl.ds(0, L)] = jnp.full((L,), ta, jnp.int32)
        pltpu.sync_copy(cntv, ca_hbm.at[pl.ds(wid * L, L)])
        cntv[pl.ds(0, L)] = jnp.full((L,), tb, jnp.int32)
        pltpu.sync_copy(cntv, cb_hbm.at[pl.ds(wid * L, L)])
        _merge_store(ha, hb, H2, hist_hbm, wid * H2)

    def compact_pass(in_shift, out_shift, hist_size, lut_size,
                     packed_lut, write_stream, two_streams):
        out_type = [jax.ShapeDtypeStruct((NW * hist_size,), jnp.int32)]
        if write_stream:
            out_type += [
                jax.ShapeDtypeStruct((N,), jnp.int32),
                jax.ShapeDtypeStruct((NW * L,), jnp.int32),
            ]
        scratch = [
            pltpu.VMEM((lut_size,), jnp.int32),
            pltpu.VMEM((hist_size,), jnp.int32),
            pltpu.VMEM((CH,), jnp.int32),
            pltpu.VMEM((L,), jnp.int32),
        ]
        if write_stream:
            scratch += [pltpu.VMEM((CH + L,), jnp.int32),
                        pltpu.VMEM((L,), jnp.int32)]
        dig_bits = in_shift - out_shift
        lut_entries = lut_size * (2 if packed_lut else 1)
        out_mask = jnp.int32((1 << out_shift) - 1) if out_shift else None

        def body(*args):
            ns = 2 if two_streams else 1
            ins = args[:2 * ns + 1]
            if write_stream:
                (hist_hbm, str_hbm, cnt_hbm, lut, hist, buf, nv,
                 cbuf, cntv) = args[2 * ns + 1:]
            else:
                hist_hbm, lut, hist, buf, nv = args[2 * ns + 1:]
            lut_hbm = ins[2 * ns]
            wid = _wid()
            pltpu.sync_copy(lut_hbm, lut)
            _zero(hist, hist_size)
            base = wid * EPW
            iota = lax.iota(jnp.int32, L)
            neg1 = jnp.full((L,), -1, jnp.int32)

            def process_stream(s_hbm, c_hbm, tot0):
                pltpu.sync_copy(c_hbm.at[pl.ds(wid * L, L)], nv)
                nvec = nv[pl.ds(0, L)]
                n_in = jnp.max(nvec)
                nchunks = (n_in + jnp.int32(CH - 1)) // jnp.int32(CH)

                def chunk(c, total):
                    pltpu.sync_copy(s_hbm.at[pl.ds(base + c * CH, CH)], buf)
                    rem = nvec - c * jnp.int32(CH)

                    def vec(i, off):
                        for j in range(UNR):
                            gi = (i * UNR + j) * L
                            e = buf[pl.ds(gi, L)]
                            inr = (gi + iota) < rem
                            b = (lax.shift_right_logical(e, in_shift)
                                 & jnp.int32(lut_entries - 1))
                            if packed_lut:
                                word = plsc.load_gather(
                                    lut, [lax.shift_right_logical(b, 1)])
                                half = (b & jnp.int32(1)) << 4
                                v16 = (lax.shift_right_logical(word, half)
                                       & jnp.int32(0xFFFF))
                                row = v16 - jnp.int32(1)
                            else:
                                row = plsc.load_gather(lut, [b])
                            valid = (e >= 0) & inr & (row >= 0)
                            bits = (lax.shift_right_logical(e, out_shift)
                                    & jnp.int32((1 << dig_bits) - 1))
                            bn = jnp.where(valid, (row << dig_bits) + bits,
                                           jnp.int32(0))
                            cnt, last = plsc.scan_count(bn, mask=valid)
                            plsc.addupdate_scatter(hist, [bn], cnt, mask=last)
                            if write_stream:
                                packed = (bn << out_shift) | (e & out_mask)
                                vi = valid.astype(jnp.int32)
                                pos = plsc.cumsum(vi)
                                dest = off + jnp.maximum(pos - 1, jnp.int32(0))
                                plsc.store_scatter(cbuf, [dest], packed,
                                                   mask=valid)
                                off = off + jnp.sum(vi)
                        return off

                    off = lax.fori_loop(0, CH // (L * UNR), vec, jnp.int32(0))
                    if write_stream:
                        plsc.store_scatter(cbuf, [off + iota], neg1)
                        off = (off + 7) & jnp.int32(~7)
                        pltpu.sync_copy(
                            cbuf.at[pl.ds(0, CH)],
                            str_hbm.at[pl.ds(
                                pl.multiple_of(base + total, 8), CH)])
                        return total + off
                    return total

                return lax.fori_loop(0, nchunks, chunk, tot0)

            total = process_stream(ins[0], ins[1], jnp.int32(0))
            if two_streams:
                total = process_stream(ins[2], ins[3], total)
            if write_stream:
                cntv[pl.ds(0, L)] = jnp.full((L,), total, jnp.int32)
                pltpu.sync_copy(cntv, cnt_hbm.at[pl.ds(wid * L, L)])
            pltpu.sync_copy(hist,
                            hist_hbm.at[pl.ds(wid * hist_size, hist_size)])

        return functools.partial(
            pl.kernel,
            out_type=tuple(out_type) if write_stream else out_type[0],
            mesh=mesh, compiler_params=cparams,
            scratch_types=scratch)(body)

    pass3 = compact_pass(in_shift=S2, out_shift=S3, hist_size=H3,
                         lut_size=H2, packed_lut=False,
                         write_stream=True, two_streams=True)
    pass4 = compact_pass(in_shift=S3, out_shift=0, hist_size=H4,
                         lut_size=H3 // 2, packed_lut=True,
                         write_stream=False, two_streams=False)

    @functools.partial(
        pl.kernel,
        out_type=jax.ShapeDtypeStruct((NW * 6 * L,), jnp.float32),
        mesh=mesh,
        compiler_params=cparams,
        scratch_types=[
            pltpu.VMEM((CH,), jnp.float32),
            pltpu.VMEM((6 * L,), jnp.float32),
            pltpu.VMEM((2 * L,), jnp.float32),
        ],
    )
    def pass5(d_hbm, params_hbm, out_hbm, buf, obuf, pbuf):
        wid = _wid()
        pltpu.sync_copy(params_hbm, pbuf)
        fb0 = pbuf[pl.ds(0, L)]
        fbl = pbuf[pl.ds(L, L)]
        base = wid * EPW
        zf = jnp.zeros((L,), jnp.float32)

        def chunk(c, carry):
            pltpu.sync_copy(d_hbm.at[pl.ds(base + c * CH, CH)], buf)

            def vec(i, acc):
                cl, sl, ql, cr, sr, qr = acc
                zero = jnp.float32(0.0)
                for j in range(UNR):
                    v = buf[pl.ds((i * UNR + j) * L, L)]
                    ml = v < fb0
                    mr = v >= fbl
                    dl = v - fb0
                    dr = v - fbl
                    cl = cl + jnp.where(ml, jnp.float32(1.0), zero)
                    sl = sl + jnp.where(ml, dl, zero)
                    ql = ql + jnp.where(ml, dl * dl, zero)
                    cr = cr + jnp.where(mr, jnp.float32(1.0), zero)
                    sr = sr + jnp.where(mr, dr, zero)
                    qr = qr + jnp.where(mr, dr * dr, zero)
                return (cl, sl, ql, cr, sr, qr)

            return lax.fori_loop(0, CH // (L * UNR), vec, carry)

        acc = lax.fori_loop(0, NCH, chunk, (zf, zf, zf, zf, zf, zf))
        for k in range(6):
            obuf[pl.ds(k * L, L)] = acc[k]
        pltpu.sync_copy(obuf, out_hbm.at[pl.ds(wid * 6 * L, 6 * L)])

    @functools.partial(
        pl.kernel,
        out_type=jax.ShapeDtypeStruct((NQ,), jnp.float32),
        mesh=mesh,
        compiler_params=cparams,
        scratch_types=[
            pltpu.VMEM((N_BINS,), jnp.float32),
            pltpu.VMEM((N_BINS,), jnp.float32),
            pltpu.VMEM((6 * L,), jnp.float32),
            pltpu.VMEM((QPW,), jnp.float32),
            pltpu.VMEM((QPW,), jnp.float32),
        ],
    )
    def pass6(x_hbm, fb_hbm, w_hbm, params_hbm, out_hbm,
              fb, w, prm, qbuf, obuf):
        wid = _wid()
        pltpu.sync_copy(fb_hbm, fb)
        pltpu.sync_copy(w_hbm, w)
        pltpu.sync_copy(params_hbm, prm)
        base = wid * QPW
        pltpu.sync_copy(x_hbm.at[pl.ds(base, QPW)], qbuf)
        fb0 = prm[pl.ds(0, L)]
        fbl = prm[pl.ds(L, L)]
        c_l = prm[pl.ds(2 * L, L)]
        ils = prm[pl.ds(3 * L, L)]
        c_r = prm[pl.ds(4 * L, L)]
        irs = prm[pl.ds(5 * L, L)]
        inf = jnp.float32(jnp.inf)

        def vec(i, carry):
            xq = qbuf[pl.ds(i * L, L)]
            pos = jnp.zeros((L,), jnp.int32)
            for step in (512, 256, 128, 64, 32, 16, 8, 4, 2, 1):
                cand = pos + jnp.int32(step)
                vb = plsc.load_gather(fb, [cand - 1])
                pos = jnp.where(vb <= xq, cand, pos)
            wv = plsc.load_gather(w, [pos])
            hi = plsc.load_gather(fb, [pos])
            lo_idx = jnp.maximum(pos - 1, jnp.int32(0))
            lov = plsc.load_gather(fb, [lo_idx])
            lov = jnp.where(pos == 0, -inf, lov)
            density = wv / (hi - lov)
            tl = (fb0 - xq) * ils
            tr = (xq - fbl) * irs
            left = c_l * jnp.exp(jnp.float32(-0.5) * tl * tl)
            right = c_r * jnp.exp(jnp.float32(-0.5) * tr * tr)
            half = jnp.where(xq < fb0, left, right)
            mid = (xq >= fb0) & (xq < fbl)
            obuf[pl.ds(i * L, L)] = jnp.where(mid, density, half)
            return carry

        lax.fori_loop(0, QPW // L, vec, 0)
        pltpu.sync_copy(obuf, out_hbm.at[pl.ds(base, QPW)])

    return pass1, pass2, pass3, pass4, pass5, pass6


def _rows_and_lut(bins, table_size):
    isnew = jnp.concatenate(
        [jnp.ones((1,), jnp.bool_), bins[1:] != bins[:-1]])
    rows = jnp.cumsum(isnew.astype(jnp.int32)) - 1
    lut = jnp.full((table_size,), -1, jnp.int32).at[bins].set(rows)
    return rows, lut


def _split_ranks(hist2d, rows, rw):
    h = hist2d[rows]
    c = jnp.cumsum(h, axis=1)
    digit = jnp.sum((c <= rw[:, None]).astype(jnp.int32), axis=1)
    excl = c - h
    rw_new = rw - jnp.take_along_axis(excl, digit[:, None], axis=1)[:, 0]
    return digit.astype(jnp.int32), rw_new


def kernel(weights, prior_samples, x):
    pass1, pass2, pass3, pass4, pass5, pass6 = _build()
    d = prior_samples
    ranks = jnp.int32(N // N_BINS) * jnp.arange(1, N_BINS, dtype=jnp.int32)

    hist1 = pass1(d).reshape(NW, H1).sum(axis=0)
    cdf1 = jnp.cumsum(hist1)
    b15 = jnp.searchsorted(cdf1, ranks, side="right").astype(jnp.int32)
    rw = ranks - (cdf1[b15] - hist1[b15])
    rows1, lut1 = _rows_and_lut(b15, H1)

    hist2, sa, sb, ca, cb = pass2(d, lut1)
    bits2, rw = _split_ranks(
        hist2.reshape(NW, N_BINS, 1 << B2x).sum(axis=0), rows1, rw)
    bin2 = (rows1 << B2x) + bits2
    rows2, lut2 = _rows_and_lut(bin2, H2)

    hist3, s3, c3 = pass3(sa, ca, sb, cb, lut2)
    bits3, rw = _split_ranks(
        hist3.reshape(NW, N_BINS, 1 << B3).sum(axis=0), rows2, rw)
    bin3 = (rows2 << B3) + bits3
    rows3, lut3 = _rows_and_lut(bin3, H3)
    lut3p16 = (lut3 + 1).reshape(H3 // 2, 2)
    lut3p = lut3p16[:, 0] | (lut3p16[:, 1] << 16)

    hist4 = pass4(s3, c3, lut3p)
    bits4, _ = _split_ranks(
        hist4.reshape(NW, N_BINS, 1 << B4).sum(axis=0), rows3, rw)

    uk = (b15 << (32 - B1)) | (bits2 << S2) | (bits3 << S3) | bits4
    key = uk ^ _I32_MIN
    ubits = key ^ (jnp.right_shift(key, 31) & _M31)
    fb = lax.bitcast_convert_type(ubits, jnp.float32)

    fb0 = fb[0]
    fbl = fb[-1]
    p5 = jnp.broadcast_to(jnp.stack([fb0, fbl])[:, None], (2, L)).reshape(-1)
    parts = pass5(d, p5).reshape(NW, 6, L).sum(axis=(0, 2))
    cl, sl, ql, cr, sr, qr = (parts[i] for i in range(6))
    scale = 1.0 / jnp.sqrt(jnp.float32(1.0 - 2.0 / jnp.pi))
    ls = jnp.sqrt(ql / cl - (sl / cl) ** 2) * scale
    rs = jnp.sqrt(qr / cr - (sr / cr) ** 2) * scale

    w = weights / weights.sum()
    s2pi = jnp.sqrt(jnp.float32(2.0) * jnp.pi)
    c_l = w[0] * 2.0 / (ls * s2pi)
    c_r = w[-1] * 2.0 / (rs * s2pi)
    prm = jnp.broadcast_to(
        jnp.stack([fb0, fbl, c_l, 1.0 / ls, c_r, 1.0 / rs])[:, None],
        (6, L)).reshape(-1)
    fbpad = jnp.concatenate([fb, jnp.full((1,), jnp.inf, jnp.float32)])
    return pass6(x, fbpad, w, prm)
